# baseline (speedup 1.0000x reference)
import functools

import jax
import jax.numpy as jnp
from jax import lax
from jax.experimental import pallas as pl
from jax.experimental.pallas import tpu as pltpu
from jax.experimental.pallas import tpu_sc as plsc

B, NP1, K = 32, 20000, 32
G = 16
C = 400
NCH = NP1 // C


def _interference_body(psi_hbm, all_hbm, knn_hbm, prm_hbm, a_hbm, d_hbm,
                       psi_v, prm_v,
                       idx0, idx1, all0, all1, a0, a1, d0, d1,
                       sr0, sr1, sw0, sw1):
    c = lax.axis_index("c")
    s = lax.axis_index("s")
    b = s * 2 + c

    pltpu.sync_copy(psi_hbm.at[pl.ds(b * (2 * NP1), 2 * NP1)], psi_v)
    pltpu.sync_copy(prm_hbm.at[pl.ds(b * 16, 16)], prm_v)
    prm = prm_v[pl.ds(0, 16)]
    qx = prm[0]
    qy = prm[1]
    lam = prm[2]
    mu2 = prm[3]
    cx = prm[4]
    cy = prm[5]

    iota = lax.broadcasted_iota(jnp.int32, (G,), 0)
    knn_base = b * (NP1 * K)
    all_base = b * (2 * NP1)
    out_base = b * NP1

    def start_reads(ci, idxb, allb, srb):
        pltpu.async_copy(knn_hbm.at[pl.ds(knn_base + ci * (C * K), C * K)],
                         idxb, srb)
        pltpu.async_copy(all_hbm.at[pl.ds(all_base + ci * (2 * C), 2 * C)],
                         allb, srb)

    start_reads(0, idx0, all0, sr0)
    start_reads(1, idx1, all1, sr1)

    bufs = ((idx0, all0, a0, d0, sr0, sw0), (idx1, all1, a1, d1, sr1, sw1))

    def outer(j, carry):
        for pb, (idxb, allb, ab, db, srb, swb) in enumerate(bufs):
            ci = 2 * j + pb
            c0 = ci * C
            pltpu.make_async_copy(
                knn_hbm.at[pl.ds(knn_base, C * K)], idxb, srb).wait()
            pltpu.make_async_copy(
                all_hbm.at[pl.ds(all_base, 2 * C)], allb, srb).wait()

            @pl.when(j > 0)
            def _():
                pltpu.make_async_copy(
                    ab, a_hbm.at[pl.ds(out_base, C)], swb).wait()
                pltpu.make_async_copy(
                    db, d_hbm.at[pl.ds(out_base, C)], swb).wait()

            def group(g, carry2, idxb=idxb, allb=allb, ab=ab, db=db, c0=c0):
                nloc = g * G
                lane_n = nloc + iota
                pos = lane_n * K
                accx = [jnp.zeros((G,), jnp.float32) for _ in range(4)]
                accy = [jnp.zeros((G,), jnp.float32) for _ in range(4)]
                for k in range(K):
                    iv = plsc.load_gather(idxb, [pos + k])
                    i2 = iv + iv
                    w = k & 3
                    accx[w] = accx[w] + plsc.load_gather(psi_v, [i2])
                    accy[w] = accy[w] + plsc.load_gather(psi_v, [i2 + 1])
                ax = (accx[0] + accx[1]) + (accx[2] + accx[3])
                ay = (accy[0] + accy[1]) + (accy[2] + accy[3])
                nv2 = (c0 + lane_n) * 2
                px = plsc.load_gather(psi_v, [nv2])
                py = plsc.load_gather(psi_v, [nv2 + 1])
                av = px * (qx + lam * ax) + py * (qy + lam * ay)
                lv2 = lane_n * 2
                wx = plsc.load_gather(allb, [lv2]) - cx
                wy = plsc.load_gather(allb, [lv2 + 1]) - cy
                dv = mu2 * (wx * wx + wy * wy)
                ab[pl.ds(nloc, G)] = av
                db[pl.ds(nloc, G)] = dv
                return carry2

            lax.fori_loop(0, C // G, group, 0)

            pltpu.async_copy(ab, a_hbm.at[pl.ds(out_base + c0, C)], swb)
            pltpu.async_copy(db, d_hbm.at[pl.ds(out_base + c0, C)], swb)

            @pl.when(j < NCH // 2 - 1)
            def _():
                start_reads(ci + 2, idxb, allb, srb)
        return carry

    lax.fori_loop(0, NCH // 2, outer, 0)

    for (idxb, allb, ab, db, srb, swb) in bufs:
        pltpu.make_async_copy(ab, a_hbm.at[pl.ds(out_base, C)], swb).wait()
        pltpu.make_async_copy(db, d_hbm.at[pl.ds(out_base, C)], swb).wait()


def _interference_sc(psi_flat, all_flat, knn_flat, prm):
    mesh = plsc.VectorSubcoreMesh(core_axis_name="c", subcore_axis_name="s")
    fn = functools.partial(
        pl.kernel,
        out_type=(jax.ShapeDtypeStruct((B * NP1,), jnp.float32),
                  jax.ShapeDtypeStruct((B * NP1,), jnp.float32)),
        mesh=mesh,
        scratch_types=[
            pltpu.VMEM((2 * NP1,), jnp.float32),
            pltpu.VMEM((16,), jnp.float32),
            pltpu.VMEM((C * K,), jnp.int32),
            pltpu.VMEM((C * K,), jnp.int32),
            pltpu.VMEM((2 * C,), jnp.float32),
            pltpu.VMEM((2 * C,), jnp.float32),
            pltpu.VMEM((C,), jnp.float32),
            pltpu.VMEM((C,), jnp.float32),
            pltpu.VMEM((C,), jnp.float32),
            pltpu.VMEM((C,), jnp.float32),
            pltpu.SemaphoreType.DMA,
            pltpu.SemaphoreType.DMA,
            pltpu.SemaphoreType.DMA,
            pltpu.SemaphoreType.DMA,
        ],
        compiler_params=pltpu.CompilerParams(needs_layout_passes=False),
    )(_interference_body)
    return fn(psi_flat, all_flat, knn_flat, prm)


def _epilogue_body(a_ref, d_ref, mask_ref, out_ref):
    a = a_ref[0, :, :]
    d = d_ref[0, :, :]
    mk = mask_ref[0, :, :]
    scores = a - jnp.sqrt(d)
    scores = jnp.where(mk > 0.5, jnp.float32(-1e9), scores)
    m = jnp.max(scores)
    e = jnp.exp(scores - m)
    out_ref[0, :, :] = scores - m - jnp.log(jnp.sum(e))


def _epilogue_tc(a, d, maskf):
    return pl.pallas_call(
        _epilogue_body,
        grid=(B,),
        in_specs=[
            pl.BlockSpec((1, 1, NP1), lambda i: (i, 0, 0)),
            pl.BlockSpec((1, 1, NP1), lambda i: (i, 0, 0)),
            pl.BlockSpec((1, 1, NP1), lambda i: (i, 0, 0)),
        ],
        out_specs=pl.BlockSpec((1, 1, NP1), lambda i: (i, 0, 0)),
        out_shape=jax.ShapeDtypeStruct((B, 1, NP1), jnp.float32),
    )(a, d, maskf)


def kernel(query, psi_prime, knn_indices, mask, current_coords, all_coords, lam, mu):
    prm = jnp.zeros((B, 16), jnp.float32)
    prm = prm.at[:, 0].set(query[:, 0])
    prm = prm.at[:, 1].set(query[:, 1])
    prm = prm.at[:, 2].set(lam)
    prm = prm.at[:, 3].set(mu * mu)
    prm = prm.at[:, 4].set(current_coords[:, 0])
    prm = prm.at[:, 5].set(current_coords[:, 1])
    a, d = _interference_sc(psi_prime.reshape(B * NP1 * 2),
                            all_coords.reshape(B * NP1 * 2),
                            knn_indices.reshape(B * NP1 * K),
                            prm.reshape(B * 16))
    maskf = mask.astype(jnp.float32)
    out = _epilogue_tc(a.reshape(B, 1, NP1), d.reshape(B, 1, NP1),
                       maskf.reshape(B, 1, NP1))
    return out.reshape(B, NP1)


# speedup vs baseline: 8.1795x; 8.1795x over previous
import functools

import jax
import jax.numpy as jnp
from jax import lax
from jax.experimental import pallas as pl
from jax.experimental.pallas import tpu as pltpu
from jax.experimental.pallas import tpu_sc as plsc

B, NP1, K = 32, 20000, 32
G = 16
C = 640
NFULL = 30
CT = NP1 - NFULL * C


def _interference_body(psi_hbm, all_hbm, knn_hbm, prm_hbm, a_hbm, d_hbm,
                       psi_v, prm_v,
                       idx0, idx1, al0, al1, a0, a1, d0, d1,
                       idxT, alT, aT, dT,
                       sr0, sr1, srT, sw0, sw1):
    c = lax.axis_index("c")
    s = lax.axis_index("s")
    b = s * 2 + c

    pltpu.sync_copy(psi_hbm.at[b], psi_v)
    pltpu.sync_copy(prm_hbm.at[pl.ds(b * 16, 16)], prm_v)
    prm = prm_v[pl.ds(0, 16)]
    qx = prm[0]
    qy = prm[1]
    lam = prm[2]
    mu2 = prm[3]
    cx = prm[4]
    cy = prm[5]

    out_base = b * NP1

    def start_reads(c0, cw, idxb, alb, srb):
        for kb in range(K // 8):
            pltpu.async_copy(knn_hbm.at[b, pl.ds(kb * 8, 8), pl.ds(c0, cw)],
                             idxb.at[pl.ds(kb * 8, 8)], srb)
        pltpu.async_copy(all_hbm.at[b, :, pl.ds(c0, cw)], alb, srb)

    def drain_reads(cw, idxb, alb, srb):
        off = 0 if cw % 128 == 0 else NP1 - cw
        pltpu.make_async_copy(
            knn_hbm.at[0, :, pl.ds(off, cw)], idxb, srb).wait()
        pltpu.make_async_copy(
            all_hbm.at[0, :, pl.ds(off, cw)], alb, srb).wait()

    def drain_writes(cw, ab, db, swb):
        pltpu.make_async_copy(ab, a_hbm.at[pl.ds(out_base, cw)], swb).wait()
        pltpu.make_async_copy(db, d_hbm.at[pl.ds(out_base, cw)], swb).wait()

    def compute_chunk(c0, cw, idxb, alb, ab, db):
        def group(g, carry, idxb=idxb, alb=alb, ab=ab, db=db, c0=c0):
            nloc = g * G
            accx = [jnp.zeros((G,), jnp.float32) for _ in range(4)]
            accy = [jnp.zeros((G,), jnp.float32) for _ in range(4)]
            for k in range(K):
                iv = idxb[k, pl.ds(nloc, G)]
                w = k & 3
                accx[w] = accx[w] + plsc.load_gather(psi_v, [jnp.zeros((G,), jnp.int32), iv])
                accy[w] = accy[w] + plsc.load_gather(psi_v, [jnp.ones((G,), jnp.int32), iv])
            ax = (accx[0] + accx[1]) + (accx[2] + accx[3])
            ay = (accy[0] + accy[1]) + (accy[2] + accy[3])
            px = psi_v[0, pl.ds(c0 + nloc, G)]
            py = psi_v[1, pl.ds(c0 + nloc, G)]
            av = px * (qx + lam * ax) + py * (qy + lam * ay)
            wx = alb[0, pl.ds(nloc, G)] - cx
            wy = alb[1, pl.ds(nloc, G)] - cy
            dv = mu2 * (wx * wx + wy * wy)
            ab[pl.ds(nloc, G)] = av
            db[pl.ds(nloc, G)] = dv
            return carry

        lax.fori_loop(0, cw // G, group, 0)

    def write_chunk(c0, cw, ab, db, swb):
        pltpu.async_copy(ab, a_hbm.at[pl.ds(out_base + c0, cw)], swb)
        pltpu.async_copy(db, d_hbm.at[pl.ds(out_base + c0, cw)], swb)

    bufs = ((idx0, al0, a0, d0, sr0, sw0), (idx1, al1, a1, d1, sr1, sw1))

    for pb, (idxb, alb, ab, db, srb, swb) in enumerate(bufs):
        start_reads(pb * C, C, idxb, alb, srb)

    def outer(j, carry):
        for pb, (idxb, alb, ab, db, srb, swb) in enumerate(bufs):
            ci = 2 * j + pb
            c0 = ci * C
            drain_reads(C, idxb, alb, srb)

            @pl.when(j > 0)
            def _():
                drain_writes(C, ab, db, swb)

            compute_chunk(c0, C, idxb, alb, ab, db)
            write_chunk(c0, C, ab, db, swb)

            @pl.when(ci + 2 < NFULL)
            def _():
                start_reads((ci + 2) * C, C, idxb, alb, srb)

            if pb == 0:
                @pl.when(j == NFULL // 2 - 1)
                def _():
                    start_reads(NFULL * C, CT, idxT, alT, srT)
        return carry

    lax.fori_loop(0, NFULL // 2, outer, 0)

    drain_reads(CT, idxT, alT, srT)
    drain_writes(C, a0, d0, sw0)
    compute_chunk(NFULL * C, CT, idxT, alT, aT, dT)
    write_chunk(NFULL * C, CT, aT, dT, sw0)
    drain_writes(C, a1, d1, sw1)
    pltpu.make_async_copy(aT, a_hbm.at[pl.ds(out_base, CT)], sw0).wait()
    pltpu.make_async_copy(dT, d_hbm.at[pl.ds(out_base, CT)], sw0).wait()


def _interference_sc(psi_t, all_t, knn_t, prm):
    mesh = plsc.VectorSubcoreMesh(core_axis_name="c", subcore_axis_name="s")
    fn = functools.partial(
        pl.kernel,
        out_type=(jax.ShapeDtypeStruct((B * NP1,), jnp.float32),
                  jax.ShapeDtypeStruct((B * NP1,), jnp.float32)),
        mesh=mesh,
        scratch_types=[
            pltpu.VMEM((2, NP1), jnp.float32),
            pltpu.VMEM((16,), jnp.float32),
            pltpu.VMEM((K, C), jnp.int32),
            pltpu.VMEM((K, C), jnp.int32),
            pltpu.VMEM((2, C), jnp.float32),
            pltpu.VMEM((2, C), jnp.float32),
            pltpu.VMEM((C,), jnp.float32),
            pltpu.VMEM((C,), jnp.float32),
            pltpu.VMEM((C,), jnp.float32),
            pltpu.VMEM((C,), jnp.float32),
            pltpu.VMEM((K, CT), jnp.int32),
            pltpu.VMEM((2, CT), jnp.float32),
            pltpu.VMEM((CT,), jnp.float32),
            pltpu.VMEM((CT,), jnp.float32),
            pltpu.SemaphoreType.DMA,
            pltpu.SemaphoreType.DMA,
            pltpu.SemaphoreType.DMA,
            pltpu.SemaphoreType.DMA,
            pltpu.SemaphoreType.DMA,
        ],
        compiler_params=pltpu.CompilerParams(needs_layout_passes=False),
    )(_interference_body)
    return fn(psi_t, all_t, knn_t, prm)


def _epilogue_body(a_ref, d_ref, mask_ref, out_ref):
    a = a_ref[...]
    d = d_ref[...]
    mk = mask_ref[...]
    scores = a - jnp.sqrt(d)
    scores = jnp.where(mk > 0.5, jnp.float32(-1e9), scores)
    m = jnp.max(scores, axis=-1, keepdims=True)
    e = jnp.exp(scores - m)
    ssum = jnp.sum(e, axis=-1, keepdims=True)
    out_ref[...] = scores - m - jnp.log(ssum)


def _epilogue_tc(a, d, maskf):
    return pl.pallas_call(
        _epilogue_body,
        grid=(B // 8,),
        in_specs=[
            pl.BlockSpec((8, NP1), lambda i: (i, 0)),
            pl.BlockSpec((8, NP1), lambda i: (i, 0)),
            pl.BlockSpec((8, NP1), lambda i: (i, 0)),
        ],
        out_specs=pl.BlockSpec((8, NP1), lambda i: (i, 0)),
        out_shape=jax.ShapeDtypeStruct((B, NP1), jnp.float32),
    )(a, d, maskf)


def kernel(query, psi_prime, knn_indices, mask, current_coords, all_coords, lam, mu):
    prm = jnp.zeros((B, 16), jnp.float32)
    prm = prm.at[:, 0].set(query[:, 0])
    prm = prm.at[:, 1].set(query[:, 1])
    prm = prm.at[:, 2].set(lam)
    prm = prm.at[:, 3].set(mu * mu)
    prm = prm.at[:, 4].set(current_coords[:, 0])
    prm = prm.at[:, 5].set(current_coords[:, 1])
    psi_t = psi_prime.transpose(0, 2, 1)
    all_t = all_coords.transpose(0, 2, 1)
    knn_t = knn_indices.transpose(0, 2, 1)
    a, d = _interference_sc(psi_t, all_t, knn_t, prm.reshape(B * 16))
    maskf = mask.astype(jnp.float32)
    out = _epilogue_tc(a.reshape(B, NP1), d.reshape(B, NP1), maskf)
    return out


# speedup vs baseline: 10.7370x; 1.3127x over previous
import functools

import jax
import jax.numpy as jnp
from jax import lax
from jax.experimental import pallas as pl
from jax.experimental.pallas import tpu as pltpu
from jax.experimental.pallas import tpu_sc as plsc

B, NP1, K = 32, 20000, 32
G = 16
C = 640
NFULL = 30
CT = NP1 - NFULL * C


def _interference_body(psi_hbm, all_hbm, knn_hbm, prm_hbm, a_hbm, d_hbm,
                       psi_v, prm_v,
                       idx0, idx1, al0, al1, a0, a1, d0, d1,
                       idxT, alT, aT, dT,
                       sr0, sr1, srT, sw0, sw1):
    c = lax.axis_index("c")
    s = lax.axis_index("s")
    b = s * 2 + c

    pltpu.sync_copy(psi_hbm.at[pl.ds(b * NP1, NP1)], psi_v)
    pltpu.sync_copy(prm_hbm.at[pl.ds(b * 16, 16)], prm_v)
    prm = prm_v[pl.ds(0, 16)]
    qx = prm[0]
    qy = prm[1]
    lam = prm[2]
    mu2 = prm[3]
    cx = prm[4]
    cy = prm[5]

    out_base = b * NP1

    def start_reads(c0, cw, idxb, alb, srb):
        for kb in range(K // 8):
            pltpu.async_copy(knn_hbm.at[b, pl.ds(kb * 8, 8), pl.ds(c0, cw)],
                             idxb.at[pl.ds(kb * 8, 8)], srb)
        pltpu.async_copy(all_hbm.at[b, :, pl.ds(c0, cw)], alb, srb)

    def drain_reads(cw, idxb, alb, srb):
        off = 0 if cw % 128 == 0 else NP1 - cw
        pltpu.make_async_copy(
            knn_hbm.at[0, :, pl.ds(off, cw)], idxb, srb).wait()
        pltpu.make_async_copy(
            all_hbm.at[0, :, pl.ds(off, cw)], alb, srb).wait()

    def drain_writes(cw, ab, db, swb):
        pltpu.make_async_copy(ab, a_hbm.at[pl.ds(out_base, cw)], swb).wait()
        pltpu.make_async_copy(db, d_hbm.at[pl.ds(out_base, cw)], swb).wait()

    def compute_chunk(c0, cw, idxb, alb, ab, db):
        def group(g, carry, idxb=idxb, alb=alb, ab=ab, db=db, c0=c0):
            nloc = g * G
            accx = [jnp.zeros((G,), jnp.float32) for _ in range(4)]
            accy = [jnp.zeros((G,), jnp.float32) for _ in range(4)]
            for k in range(K):
                iv = idxb[k, pl.ds(nloc, G)]
                w = k & 3
                pv = plsc.load_gather(psi_v, [iv])
                xv, yv = plsc.unpack(plsc.bitcast(pv, jnp.bfloat16),
                                     format=plsc.PackFormat.INTERLEAVED)
                accx[w] = accx[w] + xv
                accy[w] = accy[w] + yv
            ax = (accx[0] + accx[1]) + (accx[2] + accx[3])
            ay = (accy[0] + accy[1]) + (accy[2] + accy[3])
            ppv = psi_v[pl.ds(c0 + nloc, G)]
            px, py = plsc.unpack(plsc.bitcast(ppv, jnp.bfloat16),
                                 format=plsc.PackFormat.INTERLEAVED)
            av = px * (qx + lam * ax) + py * (qy + lam * ay)
            wx = alb[0, pl.ds(nloc, G)] - cx
            wy = alb[1, pl.ds(nloc, G)] - cy
            dv = mu2 * (wx * wx + wy * wy)
            ab[pl.ds(nloc, G)] = av
            db[pl.ds(nloc, G)] = dv
            return carry

        lax.fori_loop(0, cw // G, group, 0)

    def write_chunk(c0, cw, ab, db, swb):
        pltpu.async_copy(ab, a_hbm.at[pl.ds(out_base + c0, cw)], swb)
        pltpu.async_copy(db, d_hbm.at[pl.ds(out_base + c0, cw)], swb)

    bufs = ((idx0, al0, a0, d0, sr0, sw0), (idx1, al1, a1, d1, sr1, sw1))

    for pb, (idxb, alb, ab, db, srb, swb) in enumerate(bufs):
        start_reads(pb * C, C, idxb, alb, srb)

    def outer(j, carry):
        for pb, (idxb, alb, ab, db, srb, swb) in enumerate(bufs):
            ci = 2 * j + pb
            c0 = ci * C
            drain_reads(C, idxb, alb, srb)

            @pl.when(j > 0)
            def _():
                drain_writes(C, ab, db, swb)

            compute_chunk(c0, C, idxb, alb, ab, db)
            write_chunk(c0, C, ab, db, swb)

            @pl.when(ci + 2 < NFULL)
            def _():
                start_reads((ci + 2) * C, C, idxb, alb, srb)

            if pb == 0:
                @pl.when(j == NFULL // 2 - 1)
                def _():
                    start_reads(NFULL * C, CT, idxT, alT, srT)
        return carry

    lax.fori_loop(0, NFULL // 2, outer, 0)

    drain_reads(CT, idxT, alT, srT)
    drain_writes(C, a0, d0, sw0)
    compute_chunk(NFULL * C, CT, idxT, alT, aT, dT)
    write_chunk(NFULL * C, CT, aT, dT, sw0)
    drain_writes(C, a1, d1, sw1)
    pltpu.make_async_copy(aT, a_hbm.at[pl.ds(out_base, CT)], sw0).wait()
    pltpu.make_async_copy(dT, d_hbm.at[pl.ds(out_base, CT)], sw0).wait()


def _interference_sc(psi_t, all_t, knn_t, prm):
    mesh = plsc.VectorSubcoreMesh(core_axis_name="c", subcore_axis_name="s")
    fn = functools.partial(
        pl.kernel,
        out_type=(jax.ShapeDtypeStruct((B * NP1,), jnp.float32),
                  jax.ShapeDtypeStruct((B * NP1,), jnp.float32)),
        mesh=mesh,
        scratch_types=[
            pltpu.VMEM((NP1,), jnp.int32),
            pltpu.VMEM((16,), jnp.float32),
            pltpu.VMEM((K, C), jnp.int32),
            pltpu.VMEM((K, C), jnp.int32),
            pltpu.VMEM((2, C), jnp.float32),
            pltpu.VMEM((2, C), jnp.float32),
            pltpu.VMEM((C,), jnp.float32),
            pltpu.VMEM((C,), jnp.float32),
            pltpu.VMEM((C,), jnp.float32),
            pltpu.VMEM((C,), jnp.float32),
            pltpu.VMEM((K, CT), jnp.int32),
            pltpu.VMEM((2, CT), jnp.float32),
            pltpu.VMEM((CT,), jnp.float32),
            pltpu.VMEM((CT,), jnp.float32),
            pltpu.SemaphoreType.DMA,
            pltpu.SemaphoreType.DMA,
            pltpu.SemaphoreType.DMA,
            pltpu.SemaphoreType.DMA,
            pltpu.SemaphoreType.DMA,
        ],
        compiler_params=pltpu.CompilerParams(needs_layout_passes=False),
    )(_interference_body)
    return fn(psi_t, all_t, knn_t, prm)


def _epilogue_body(a_ref, d_ref, mask_ref, out_ref):
    a = a_ref[...]
    d = d_ref[...]
    mk = mask_ref[...]
    scores = a - jnp.sqrt(d)
    scores = jnp.where(mk > 0.5, jnp.float32(-1e9), scores)
    m = jnp.max(scores, axis=-1, keepdims=True)
    e = jnp.exp(scores - m)
    ssum = jnp.sum(e, axis=-1, keepdims=True)
    out_ref[...] = scores - m - jnp.log(ssum)


def _epilogue_tc(a, d, maskf):
    return pl.pallas_call(
        _epilogue_body,
        grid=(B // 8,),
        in_specs=[
            pl.BlockSpec((8, NP1), lambda i: (i, 0)),
            pl.BlockSpec((8, NP1), lambda i: (i, 0)),
            pl.BlockSpec((8, NP1), lambda i: (i, 0)),
        ],
        out_specs=pl.BlockSpec((8, NP1), lambda i: (i, 0)),
        out_shape=jax.ShapeDtypeStruct((B, NP1), jnp.float32),
    )(a, d, maskf)


def kernel(query, psi_prime, knn_indices, mask, current_coords, all_coords, lam, mu):
    zero = jnp.zeros((B,), jnp.float32)
    prm = jnp.stack(
        [query[:, 0], query[:, 1],
         jnp.broadcast_to(lam, (B,)), jnp.broadcast_to(mu * mu, (B,)),
         current_coords[:, 0], current_coords[:, 1]]
        + [zero] * 10, axis=1)
    psi_pack = jax.lax.bitcast_convert_type(
        psi_prime.astype(jnp.bfloat16), jnp.int32).reshape(B * NP1)
    all_t = all_coords.transpose(0, 2, 1)
    knn_t = knn_indices.transpose(0, 2, 1)
    a, d = _interference_sc(psi_pack, all_t, knn_t, prm.reshape(B * 16))
    maskf = mask.astype(jnp.float32)
    out = _epilogue_tc(a.reshape(B, NP1), d.reshape(B, NP1), maskf)
    return out
